# Initial kernel scaffold; baseline (speedup 1.0000x reference)
import jax
import jax.numpy as jnp
from jax.experimental import pallas as pl


def kernel(x, edge_index_0, edge_index_1, W_l0, b_l0, W_r0, W_l1, b_l1, W_r1, W_post, b_post):
    raise NotImplementedError("write your pallas kernel here")


# speedup vs baseline: 5.2820x; 5.2820x over previous
import dataclasses
import functools

import jax
import jax.numpy as jnp
from jax import lax
from jax.experimental import pallas as pl
from jax.experimental.pallas import tpu as pltpu
from jax.experimental.pallas import tpu_sc as plsc

N = 10000
E = 160000
D_IN = 256
H = 256
D_OUT = 128

NC = 2
NS = 16
CHUNK = 64
CH = 160
IDXB = 8
E_PAD = NS * CH * CHUNK
N_PAD = 10240
ROWS_PER_SUB = N_PAD // NS
HALF = 128
ZROWS = 128


RED = 128


def _sc_agg_body(xlo_hbm, xhi_hbm, src_hbm, dst_hbm, z128_hbm,
                 agglo_hbm, agghi_hbm, cnt_hbm,
                 acc_sp, cnt_stage, src_v, dst_v, gb0, gb1, cnt_priv,
                 redbuf, outcol, sem0, sem1):
    cid = lax.axis_index("c")
    sid = lax.axis_index("s")
    base = sid * ROWS_PER_SUB

    @pl.loop(0, ROWS_PER_SUB // ZROWS)
    def _(i):
        pltpu.sync_copy(z128_hbm, acc_sp.at[pl.ds(base + i * ZROWS, ZROWS)])

    zv = jnp.zeros((16,), jnp.float32)

    @pl.loop(0, N_PAD, step=16)
    def _(i):
        cnt_priv[pl.ds(i, 16)] = zv

    plsc.subcore_barrier()

    ones_v = jnp.ones((16,), jnp.float32)

    def run(x_hbm, with_counts):
        gbs = (gb0, gb1)
        sems = (sem0, sem1)

        @pl.loop(0, CH, step=IDXB)
        def _(g):
            pltpu.sync_copy(src_hbm.at[sid, pl.ds(g, IDXB)], src_v)
            pltpu.sync_copy(dst_hbm.at[sid, pl.ds(g, IDXB)], dst_v)
            cps = [None] * IDXB
            cps[0] = pltpu.async_copy(x_hbm.at[src_v.at[0]], gbs[0], sems[0])
            for j in range(IDXB):
                if j + 1 < IDXB:
                    cps[j + 1] = pltpu.async_copy(
                        x_hbm.at[src_v.at[j + 1]], gbs[(j + 1) % 2],
                        sems[(j + 1) % 2])
                if with_counts:
                    for l in range(CHUNK // 16):
                        dvec = dst_v[j, pl.ds(l * 16, 16)]
                        plsc.addupdate_scatter(cnt_priv, [dvec], ones_v)
                cps[j].wait()
                pltpu.sync_copy(gbs[j % 2], acc_sp.at[dst_v.at[j]], add=True)

    @pl.when(cid == 0)
    def _():
        run(xlo_hbm, True)

    @pl.when(cid == 1)
    def _():
        run(xhi_hbm, False)

    plsc.subcore_barrier()

    sl = pl.ds(base, ROWS_PER_SUB)

    @pl.when(cid == 1)
    def _():
        pltpu.sync_copy(acc_sp.at[sl], agghi_hbm.at[sl])

    @pl.when(cid == 0)
    def _():
        pltpu.sync_copy(acc_sp.at[sl], agglo_hbm.at[sl])
        pltpu.sync_copy(cnt_priv, cnt_stage.at[sid])
        plsc.subcore_barrier()

        for half in range(ROWS_PER_SUB // RED):
            off = base + half * RED
            pltpu.sync_copy(cnt_stage.at[:, pl.ds(off, RED)], redbuf)

            @pl.loop(0, RED, step=16)
            def _(p):
                acc = redbuf[0, pl.ds(p, 16)]
                for r in range(1, NS):
                    acc = acc + redbuf[r, pl.ds(p, 16)]
                outcol[pl.ds(p, 16)] = acc

            pltpu.sync_copy(outcol, cnt_hbm.at[pl.ds(off, RED)])


def _sc_agg(x_lo, x_hi, src3, dst3, z128):
    mesh = plsc.VectorSubcoreMesh(core_axis_name="c", subcore_axis_name="s")
    f32 = jnp.float32
    out_type = (
        jax.ShapeDtypeStruct((N_PAD, HALF), f32),
        jax.ShapeDtypeStruct((N_PAD, HALF), f32),
        jax.ShapeDtypeStruct((N_PAD,), f32),
    )
    scratch = [
        pltpu.VMEM_SHARED((N_PAD, HALF), f32),
        pltpu.VMEM_SHARED((NS, N_PAD), f32),
        pltpu.VMEM((IDXB, CHUNK), jnp.int32),
        pltpu.VMEM((IDXB, CHUNK), jnp.int32),
        pltpu.VMEM((CHUNK, HALF), f32),
        pltpu.VMEM((CHUNK, HALF), f32),
        pltpu.VMEM((N_PAD,), f32),
        pltpu.VMEM((NS, RED), f32),
        pltpu.VMEM((RED,), f32),
        pltpu.SemaphoreType.DMA,
        pltpu.SemaphoreType.DMA,
    ]
    cp = pltpu.CompilerParams()
    if "needs_layout_passes" in pltpu.CompilerParams.__dataclass_fields__:
        cp = dataclasses.replace(cp, needs_layout_passes=False)
    k = pl.kernel(_sc_agg_body, out_type=out_type, mesh=mesh,
                  scratch_types=scratch, compiler_params=cp)
    return k(x_lo, x_hi, src3, dst3, z128)


BLK = 1024
GRID = N_PAD // BLK


def _mm_bias_body(x_ref, w_ref, b_ref, o_ref):
    o_ref[...] = jnp.dot(x_ref[...], w_ref[...],
                         preferred_element_type=jnp.float32) + b_ref[...]


def _mm_bias(x, wT, b):
    kdim, m = wT.shape
    return pl.pallas_call(
        _mm_bias_body,
        grid=(GRID,),
        in_specs=[
            pl.BlockSpec((BLK, kdim), lambda i: (i, 0)),
            pl.BlockSpec((kdim, m), lambda i: (0, 0)),
            pl.BlockSpec((1, m), lambda i: (0, 0)),
        ],
        out_specs=pl.BlockSpec((BLK, m), lambda i: (i, 0)),
        out_shape=jax.ShapeDtypeStruct((N_PAD, m), jnp.float32),
    )(x, wT, b)


def _mm2_bias_body(xa_ref, xb_ref, w_ref, b_ref, o_ref):
    w = w_ref[...]
    o_ref[...] = (
        jnp.dot(xa_ref[...], w[:HALF], preferred_element_type=jnp.float32)
        + jnp.dot(xb_ref[...], w[HALF:], preferred_element_type=jnp.float32)
        + b_ref[...]
    )


def _mm2_bias(xa, xb, wT, b):
    _, m = wT.shape
    return pl.pallas_call(
        _mm2_bias_body,
        grid=(GRID,),
        in_specs=[
            pl.BlockSpec((BLK, HALF), lambda i: (i, 0)),
            pl.BlockSpec((BLK, HALF), lambda i: (i, 0)),
            pl.BlockSpec((2 * HALF, m), lambda i: (0, 0)),
            pl.BlockSpec((1, m), lambda i: (0, 0)),
        ],
        out_specs=pl.BlockSpec((BLK, m), lambda i: (i, 0)),
        out_shape=jax.ShapeDtypeStruct((N_PAD, m), jnp.float32),
    )(xa, xb, wT, b)


def _layer_body(alo_ref, ahi_ref, cnt_ref, xr_ref, w_ref, olo_ref, ohi_ref):
    inv = 1.0 / jnp.maximum(cnt_ref[...], 1.0)
    w = w_ref[...]
    h = (
        jnp.dot(alo_ref[...] * inv, w[:HALF],
                preferred_element_type=jnp.float32)
        + jnp.dot(ahi_ref[...] * inv, w[HALF:],
                  preferred_element_type=jnp.float32)
        + xr_ref[...]
    )
    h = jnp.maximum(h, 0.0)
    olo_ref[...] = h[:, :HALF]
    ohi_ref[...] = h[:, HALF:]


def _layer0(agg_lo, agg_hi, cnt, xr, wlT):
    return pl.pallas_call(
        _layer_body,
        grid=(GRID,),
        in_specs=[
            pl.BlockSpec((BLK, HALF), lambda i: (i, 0)),
            pl.BlockSpec((BLK, HALF), lambda i: (i, 0)),
            pl.BlockSpec((BLK, 1), lambda i: (i, 0)),
            pl.BlockSpec((BLK, H), lambda i: (i, 0)),
            pl.BlockSpec((H, H), lambda i: (0, 0)),
        ],
        out_specs=[
            pl.BlockSpec((BLK, HALF), lambda i: (i, 0)),
            pl.BlockSpec((BLK, HALF), lambda i: (i, 0)),
        ],
        out_shape=[
            jax.ShapeDtypeStruct((N_PAD, HALF), jnp.float32),
            jax.ShapeDtypeStruct((N_PAD, HALF), jnp.float32),
        ],
    )(agg_lo, agg_hi, cnt, xr, wlT)


def _final_body(h0lo_ref, h0hi_ref, alo_ref, ahi_ref, cnt_ref, xr_ref,
                wl_ref, wpa_ref, wpb_ref, bp_ref, o_ref):
    inv = 1.0 / jnp.maximum(cnt_ref[...], 1.0)
    wl = wl_ref[...]
    h1 = (
        jnp.dot(alo_ref[...] * inv, wl[:HALF],
                preferred_element_type=jnp.float32)
        + jnp.dot(ahi_ref[...] * inv, wl[HALF:],
                  preferred_element_type=jnp.float32)
        + xr_ref[...]
    )
    wpa = wpa_ref[...]
    wpb = wpb_ref[...]
    logits = (
        jnp.dot(h0lo_ref[...], wpa[:HALF], preferred_element_type=jnp.float32)
        + jnp.dot(h0hi_ref[...], wpa[HALF:],
                  preferred_element_type=jnp.float32)
        + jnp.dot(h1[:, :HALF], wpb[:HALF],
                  preferred_element_type=jnp.float32)
        + jnp.dot(h1[:, HALF:], wpb[HALF:],
                  preferred_element_type=jnp.float32)
        + bp_ref[...]
    )
    m = jnp.max(logits, axis=-1, keepdims=True)
    lse = jnp.log(jnp.sum(jnp.exp(logits - m), axis=-1, keepdims=True)) + m
    o_ref[...] = logits - lse


def _final(h0_lo, h0_hi, agg_lo, agg_hi, cnt, xr1, wl1T, wpaT, wpbT, bp):
    return pl.pallas_call(
        _final_body,
        grid=(GRID,),
        in_specs=[
            pl.BlockSpec((BLK, HALF), lambda i: (i, 0)),
            pl.BlockSpec((BLK, HALF), lambda i: (i, 0)),
            pl.BlockSpec((BLK, HALF), lambda i: (i, 0)),
            pl.BlockSpec((BLK, HALF), lambda i: (i, 0)),
            pl.BlockSpec((BLK, 1), lambda i: (i, 0)),
            pl.BlockSpec((BLK, H), lambda i: (i, 0)),
            pl.BlockSpec((H, H), lambda i: (0, 0)),
            pl.BlockSpec((H, D_OUT), lambda i: (0, 0)),
            pl.BlockSpec((H, D_OUT), lambda i: (0, 0)),
            pl.BlockSpec((1, D_OUT), lambda i: (0, 0)),
        ],
        out_specs=pl.BlockSpec((BLK, D_OUT), lambda i: (i, 0)),
        out_shape=jax.ShapeDtypeStruct((N_PAD, D_OUT), jnp.float32),
    )(h0_lo, h0_hi, agg_lo, agg_hi, cnt, xr1, wl1T, wpaT, wpbT, bp)


def _prep_edges(edge_index):
    src = edge_index[0].astype(jnp.int32)
    dst = edge_index[1].astype(jnp.int32)
    pad = E_PAD - E
    r = jnp.arange(pad, dtype=jnp.int32)
    src = jnp.concatenate([src, r % N])
    dst = jnp.concatenate([dst, N + r % (N_PAD - N)])
    return src.reshape(NS, CH, CHUNK), dst.reshape(NS, CH, CHUNK)


def kernel(x, edge_index_0, edge_index_1, W_l0, b_l0, W_r0,
           W_l1, b_l1, W_r1, W_post, b_post):
    f32 = jnp.float32
    src0, dst0 = _prep_edges(edge_index_0)
    src1, dst1 = _prep_edges(edge_index_1)

    xp = jnp.pad(x, ((0, N_PAD - N), (0, 0)))
    x_lo = xp[:, :HALF]
    x_hi = xp[:, HALF:]

    z128 = jnp.zeros((ZROWS, HALF), f32)

    wl0T = W_l0.T
    wr0T = W_r0.T
    wl1T = W_l1.T
    wr1T = W_r1.T
    wpaT = W_post[:, :H].T
    wpbT = W_post[:, H:].T
    bl0 = b_l0.reshape(1, H)
    bl1 = b_l1.reshape(1, H)
    bp = b_post.reshape(1, D_OUT)

    agg0_lo, agg0_hi, cnt0 = _sc_agg(x_lo, x_hi, src0, dst0, z128)
    xr0 = _mm_bias(xp, wr0T, bl0)
    cnt0c = cnt0.reshape(N_PAD, 1)
    h0_lo, h0_hi = _layer0(agg0_lo, agg0_hi, cnt0c, xr0, wl0T)

    agg1_lo, agg1_hi, cnt1 = _sc_agg(h0_lo, h0_hi, src1, dst1, z128)
    xr1 = _mm2_bias(h0_lo, h0_hi, wr1T, bl1)
    cnt1c = cnt1.reshape(N_PAD, 1)

    out = _final(h0_lo, h0_hi, agg1_lo, agg1_hi, cnt1c, xr1, wl1T,
                 wpaT, wpbT, bp)
    return out[:N]


# speedup vs baseline: 6.6052x; 1.2505x over previous
import dataclasses
import functools

import jax
import jax.numpy as jnp
from jax import lax
from jax.experimental import pallas as pl
from jax.experimental.pallas import tpu as pltpu
from jax.experimental.pallas import tpu_sc as plsc

N = 10000
E = 160000
D_IN = 256
H = 256
D_OUT = 128

NC = 2
NS = 16
CHUNK = 128
CH = 80
IDXB = 8
E_PAD = NS * CH * CHUNK
N_PAD = 10240
ROWS_PER_SUB = N_PAD // NS
HALF = 128
ZROWS = 128


RED = 128


def _sc_agg_body(xlo_hbm, xhi_hbm, src_hbm, dst_hbm,
                 agglo_hbm, agghi_hbm, cnt_hbm, parts_hbm,
                 acc_sp, src_v, dst_v, gb0, gb1, cnt_priv,
                 redbuf, outcol, sem0, sem1):
    cid = lax.axis_index("c")
    sid = lax.axis_index("s")
    base = sid * ROWS_PER_SUB

    zv = jnp.zeros((16,), jnp.float32)

    @pl.loop(0, CHUNK)
    def _(r):
        @pl.loop(0, HALF, step=16)
        def _(c):
            gb0[r, pl.ds(c, 16)] = zv

    @pl.loop(0, N_PAD, step=16)
    def _(i):
        cnt_priv[pl.ds(i, 16)] = zv

    @pl.loop(0, ROWS_PER_SUB // ZROWS)
    def _(i):
        pltpu.sync_copy(gb0, acc_sp.at[pl.ds(base + i * ZROWS, ZROWS)])

    plsc.subcore_barrier()

    ones_v = jnp.ones((16,), jnp.float32)

    def run(x_hbm, with_counts):
        gbs = (gb0, gb1)
        sems = (sem0, sem1)

        @pl.loop(0, CH, step=IDXB)
        def _(g):
            pltpu.sync_copy(src_hbm.at[sid, pl.ds(g, IDXB)], src_v)
            pltpu.sync_copy(dst_hbm.at[sid, pl.ds(g, IDXB)], dst_v)
            cps = [None] * IDXB
            cps[0] = pltpu.async_copy(x_hbm.at[src_v.at[0]], gbs[0], sems[0])
            for j in range(IDXB):
                if j + 1 < IDXB:
                    cps[j + 1] = pltpu.async_copy(
                        x_hbm.at[src_v.at[j + 1]], gbs[(j + 1) % 2],
                        sems[(j + 1) % 2])
                if with_counts:
                    for l in range(CHUNK // 16):
                        dvec = dst_v[j, pl.ds(l * 16, 16)]
                        plsc.addupdate_scatter(cnt_priv, [dvec], ones_v)
                cps[j].wait()
                pltpu.sync_copy(gbs[j % 2], acc_sp.at[dst_v.at[j]], add=True)

    @pl.when(cid == 0)
    def _():
        run(xlo_hbm, True)

    @pl.when(cid == 1)
    def _():
        run(xhi_hbm, False)

    plsc.subcore_barrier()

    sl = pl.ds(base, ROWS_PER_SUB)

    @pl.when(cid == 1)
    def _():
        pltpu.sync_copy(acc_sp.at[sl], agghi_hbm.at[sl])

    @pl.when(cid == 0)
    def _():
        pltpu.sync_copy(acc_sp.at[sl], agglo_hbm.at[sl])
        pltpu.sync_copy(cnt_priv, parts_hbm.at[sid])
        plsc.subcore_barrier()

        @pl.loop(0, ROWS_PER_SUB // RED)
        def _(half):
            off = base + half * RED
            pltpu.sync_copy(parts_hbm.at[:, pl.ds(off, RED)], redbuf)

            @pl.loop(0, RED, step=16)
            def _(p):
                acc = redbuf[0, pl.ds(p, 16)]
                for r in range(1, NS):
                    acc = acc + redbuf[r, pl.ds(p, 16)]
                outcol[pl.ds(p, 16)] = acc

            pltpu.sync_copy(outcol, cnt_hbm.at[pl.ds(off, RED)])


def _sc_agg(x_lo, x_hi, src3, dst3):
    mesh = plsc.VectorSubcoreMesh(core_axis_name="c", subcore_axis_name="s")
    f32 = jnp.float32
    out_type = (
        jax.ShapeDtypeStruct((N_PAD, HALF), f32),
        jax.ShapeDtypeStruct((N_PAD, HALF), f32),
        jax.ShapeDtypeStruct((N_PAD,), f32),
        jax.ShapeDtypeStruct((NS, N_PAD), f32),
    )
    scratch = [
        pltpu.VMEM_SHARED((N_PAD, HALF), f32),
        pltpu.VMEM((IDXB, CHUNK), jnp.int32),
        pltpu.VMEM((IDXB, CHUNK), jnp.int32),
        pltpu.VMEM((CHUNK, HALF), f32),
        pltpu.VMEM((CHUNK, HALF), f32),
        pltpu.VMEM((N_PAD,), f32),
        pltpu.VMEM((NS, RED), f32),
        pltpu.VMEM((RED,), f32),
        pltpu.SemaphoreType.DMA,
        pltpu.SemaphoreType.DMA,
    ]
    cp = pltpu.CompilerParams()
    if "needs_layout_passes" in pltpu.CompilerParams.__dataclass_fields__:
        cp = dataclasses.replace(cp, needs_layout_passes=False)
    k = pl.kernel(_sc_agg_body, out_type=out_type, mesh=mesh,
                  scratch_types=scratch, compiler_params=cp)
    agg_lo, agg_hi, cnt, _ = k(x_lo, x_hi, src3, dst3)
    return agg_lo, agg_hi, cnt


BLK = 1024
GRID = N_PAD // BLK


def _mm_bias_body(x_ref, w_ref, b_ref, o_ref):
    o_ref[...] = jnp.dot(x_ref[...], w_ref[...],
                         preferred_element_type=jnp.float32) + b_ref[...]


def _mm_bias(x, wT, b):
    kdim, m = wT.shape
    return pl.pallas_call(
        _mm_bias_body,
        grid=(GRID,),
        in_specs=[
            pl.BlockSpec((BLK, kdim), lambda i: (i, 0)),
            pl.BlockSpec((kdim, m), lambda i: (0, 0)),
            pl.BlockSpec((1, m), lambda i: (0, 0)),
        ],
        out_specs=pl.BlockSpec((BLK, m), lambda i: (i, 0)),
        out_shape=jax.ShapeDtypeStruct((N_PAD, m), jnp.float32),
    )(x, wT, b)


def _mm2_bias_body(xa_ref, xb_ref, w_ref, b_ref, o_ref):
    w = w_ref[...]
    o_ref[...] = (
        jnp.dot(xa_ref[...], w[:HALF], preferred_element_type=jnp.float32)
        + jnp.dot(xb_ref[...], w[HALF:], preferred_element_type=jnp.float32)
        + b_ref[...]
    )


def _mm2_bias(xa, xb, wT, b):
    _, m = wT.shape
    return pl.pallas_call(
        _mm2_bias_body,
        grid=(GRID,),
        in_specs=[
            pl.BlockSpec((BLK, HALF), lambda i: (i, 0)),
            pl.BlockSpec((BLK, HALF), lambda i: (i, 0)),
            pl.BlockSpec((2 * HALF, m), lambda i: (0, 0)),
            pl.BlockSpec((1, m), lambda i: (0, 0)),
        ],
        out_specs=pl.BlockSpec((BLK, m), lambda i: (i, 0)),
        out_shape=jax.ShapeDtypeStruct((N_PAD, m), jnp.float32),
    )(xa, xb, wT, b)


def _layer_body(alo_ref, ahi_ref, cnt_ref, xr_ref, w_ref, olo_ref, ohi_ref):
    inv = 1.0 / jnp.maximum(cnt_ref[...], 1.0)
    w = w_ref[...]
    h = (
        jnp.dot(alo_ref[...] * inv, w[:HALF],
                preferred_element_type=jnp.float32)
        + jnp.dot(ahi_ref[...] * inv, w[HALF:],
                  preferred_element_type=jnp.float32)
        + xr_ref[...]
    )
    h = jnp.maximum(h, 0.0)
    olo_ref[...] = h[:, :HALF]
    ohi_ref[...] = h[:, HALF:]


def _layer0(agg_lo, agg_hi, cnt, xr, wlT):
    return pl.pallas_call(
        _layer_body,
        grid=(GRID,),
        in_specs=[
            pl.BlockSpec((BLK, HALF), lambda i: (i, 0)),
            pl.BlockSpec((BLK, HALF), lambda i: (i, 0)),
            pl.BlockSpec((BLK, 1), lambda i: (i, 0)),
            pl.BlockSpec((BLK, H), lambda i: (i, 0)),
            pl.BlockSpec((H, H), lambda i: (0, 0)),
        ],
        out_specs=[
            pl.BlockSpec((BLK, HALF), lambda i: (i, 0)),
            pl.BlockSpec((BLK, HALF), lambda i: (i, 0)),
        ],
        out_shape=[
            jax.ShapeDtypeStruct((N_PAD, HALF), jnp.float32),
            jax.ShapeDtypeStruct((N_PAD, HALF), jnp.float32),
        ],
    )(agg_lo, agg_hi, cnt, xr, wlT)


def _final_body(h0lo_ref, h0hi_ref, alo_ref, ahi_ref, cnt_ref, xr_ref,
                wl_ref, wpa_ref, wpb_ref, bp_ref, o_ref):
    inv = 1.0 / jnp.maximum(cnt_ref[...], 1.0)
    wl = wl_ref[...]
    h1 = (
        jnp.dot(alo_ref[...] * inv, wl[:HALF],
                preferred_element_type=jnp.float32)
        + jnp.dot(ahi_ref[...] * inv, wl[HALF:],
                  preferred_element_type=jnp.float32)
        + xr_ref[...]
    )
    wpa = wpa_ref[...]
    wpb = wpb_ref[...]
    logits = (
        jnp.dot(h0lo_ref[...], wpa[:HALF], preferred_element_type=jnp.float32)
        + jnp.dot(h0hi_ref[...], wpa[HALF:],
                  preferred_element_type=jnp.float32)
        + jnp.dot(h1[:, :HALF], wpb[:HALF],
                  preferred_element_type=jnp.float32)
        + jnp.dot(h1[:, HALF:], wpb[HALF:],
                  preferred_element_type=jnp.float32)
        + bp_ref[...]
    )
    m = jnp.max(logits, axis=-1, keepdims=True)
    lse = jnp.log(jnp.sum(jnp.exp(logits - m), axis=-1, keepdims=True)) + m
    o_ref[...] = logits - lse


def _final(h0_lo, h0_hi, agg_lo, agg_hi, cnt, xr1, wl1T, wpaT, wpbT, bp):
    return pl.pallas_call(
        _final_body,
        grid=(GRID,),
        in_specs=[
            pl.BlockSpec((BLK, HALF), lambda i: (i, 0)),
            pl.BlockSpec((BLK, HALF), lambda i: (i, 0)),
            pl.BlockSpec((BLK, HALF), lambda i: (i, 0)),
            pl.BlockSpec((BLK, HALF), lambda i: (i, 0)),
            pl.BlockSpec((BLK, 1), lambda i: (i, 0)),
            pl.BlockSpec((BLK, H), lambda i: (i, 0)),
            pl.BlockSpec((H, H), lambda i: (0, 0)),
            pl.BlockSpec((H, D_OUT), lambda i: (0, 0)),
            pl.BlockSpec((H, D_OUT), lambda i: (0, 0)),
            pl.BlockSpec((1, D_OUT), lambda i: (0, 0)),
        ],
        out_specs=pl.BlockSpec((BLK, D_OUT), lambda i: (i, 0)),
        out_shape=jax.ShapeDtypeStruct((N_PAD, D_OUT), jnp.float32),
    )(h0_lo, h0_hi, agg_lo, agg_hi, cnt, xr1, wl1T, wpaT, wpbT, bp)


def _prep_edges(edge_index):
    src = edge_index[0].astype(jnp.int32)
    dst = edge_index[1].astype(jnp.int32)
    pad = E_PAD - E
    r = jnp.arange(pad, dtype=jnp.int32)
    src = jnp.concatenate([src, r % N])
    dst = jnp.concatenate([dst, N + r % (N_PAD - N)])
    return src.reshape(NS, CH, CHUNK), dst.reshape(NS, CH, CHUNK)


def kernel(x, edge_index_0, edge_index_1, W_l0, b_l0, W_r0,
           W_l1, b_l1, W_r1, W_post, b_post):
    f32 = jnp.float32
    src0, dst0 = _prep_edges(edge_index_0)
    src1, dst1 = _prep_edges(edge_index_1)

    xp = jnp.pad(x, ((0, N_PAD - N), (0, 0)))
    x_lo = xp[:, :HALF]
    x_hi = xp[:, HALF:]

    wl0T = W_l0.T
    wr0T = W_r0.T
    wl1T = W_l1.T
    wr1T = W_r1.T
    wpaT = W_post[:, :H].T
    wpbT = W_post[:, H:].T
    bl0 = b_l0.reshape(1, H)
    bl1 = b_l1.reshape(1, H)
    bp = b_post.reshape(1, D_OUT)

    agg0_lo, agg0_hi, cnt0 = _sc_agg(x_lo, x_hi, src0, dst0)
    xr0 = _mm_bias(xp, wr0T, bl0)
    cnt0c = cnt0.reshape(N_PAD, 1)
    h0_lo, h0_hi = _layer0(agg0_lo, agg0_hi, cnt0c, xr0, wl0T)

    agg1_lo, agg1_hi, cnt1 = _sc_agg(h0_lo, h0_hi, src1, dst1)
    xr1 = _mm2_bias(h0_lo, h0_hi, wr1T, bl1)
    cnt1c = cnt1.reshape(N_PAD, 1)

    out = _final(h0_lo, h0_hi, agg1_lo, agg1_hi, cnt1c, xr1, wl1T,
                 wpaT, wpbT, bp)
    return out[:N]


# speedup vs baseline: 6.9948x; 1.0590x over previous
import dataclasses
import functools

import jax
import jax.numpy as jnp
from jax import lax
from jax.experimental import pallas as pl
from jax.experimental.pallas import tpu as pltpu
from jax.experimental.pallas import tpu_sc as plsc

N = 10000
E = 160000
D_IN = 256
H = 256
D_OUT = 128

NC = 2
NS = 16
CHUNK = 128
CH = 80
IDXB = 8
E_PAD = NS * CH * CHUNK
N_PAD = 10240
ROWS_PER_SUB = N_PAD // NS
HALF = 128
ZROWS = 128


RED = 128


def _sc_agg_body(xlo_hbm, xhi_hbm, src_hbm, dst_hbm,
                 agglo_hbm, agghi_hbm, parts_hbm,
                 acc_sp, src_v, dst_v, gb0, gb1, cnt_priv,
                 sem0, sem1):
    cid = lax.axis_index("c")
    sid = lax.axis_index("s")
    base = sid * ROWS_PER_SUB

    zv = jnp.zeros((16,), jnp.float32)

    @pl.loop(0, CHUNK)
    def _(r):
        @pl.loop(0, HALF, step=16)
        def _(c):
            gb0[r, pl.ds(c, 16)] = zv

    @pl.loop(0, N_PAD, step=16)
    def _(i):
        cnt_priv[pl.ds(i, 16)] = zv

    @pl.loop(0, ROWS_PER_SUB // ZROWS)
    def _(i):
        pltpu.sync_copy(gb0, acc_sp.at[pl.ds(base + i * ZROWS, ZROWS)])

    plsc.subcore_barrier()

    ones_v = jnp.ones((16,), jnp.float32)

    def run(x_hbm, with_counts):
        gbs = (gb0, gb1)
        sems = (sem0, sem1)

        @pl.loop(0, CH, step=IDXB)
        def _(g):
            pltpu.sync_copy(src_hbm.at[sid, pl.ds(g, IDXB)], src_v)
            pltpu.sync_copy(dst_hbm.at[sid, pl.ds(g, IDXB)], dst_v)
            cps = [None] * IDXB
            cps[0] = pltpu.async_copy(x_hbm.at[src_v.at[0]], gbs[0], sems[0])
            for j in range(IDXB):
                if j + 1 < IDXB:
                    cps[j + 1] = pltpu.async_copy(
                        x_hbm.at[src_v.at[j + 1]], gbs[(j + 1) % 2],
                        sems[(j + 1) % 2])
                if with_counts:
                    for l in range(CHUNK // 16):
                        dvec = dst_v[j, pl.ds(l * 16, 16)]
                        plsc.addupdate_scatter(cnt_priv, [dvec], ones_v)
                cps[j].wait()
                pltpu.sync_copy(gbs[j % 2], acc_sp.at[dst_v.at[j]], add=True)

    @pl.when(cid == 0)
    def _():
        run(xlo_hbm, True)

    @pl.when(cid == 1)
    def _():
        run(xhi_hbm, False)

    plsc.subcore_barrier()

    sl = pl.ds(base, ROWS_PER_SUB)

    @pl.when(cid == 1)
    def _():
        pltpu.sync_copy(acc_sp.at[sl], agghi_hbm.at[sl])

    @pl.when(cid == 0)
    def _():
        pltpu.sync_copy(acc_sp.at[sl], agglo_hbm.at[sl])
        pltpu.sync_copy(cnt_priv, parts_hbm.at[sid])


def _sc_agg(x_lo, x_hi, src3, dst3):
    mesh = plsc.VectorSubcoreMesh(core_axis_name="c", subcore_axis_name="s")
    f32 = jnp.float32
    out_type = (
        jax.ShapeDtypeStruct((N_PAD, HALF), f32),
        jax.ShapeDtypeStruct((N_PAD, HALF), f32),
        jax.ShapeDtypeStruct((NS, N_PAD), f32),
    )
    scratch = [
        pltpu.VMEM_SHARED((N_PAD, HALF), f32),
        pltpu.VMEM((IDXB, CHUNK), jnp.int32),
        pltpu.VMEM((IDXB, CHUNK), jnp.int32),
        pltpu.VMEM((CHUNK, HALF), f32),
        pltpu.VMEM((CHUNK, HALF), f32),
        pltpu.VMEM((N_PAD,), f32),
        pltpu.SemaphoreType.DMA,
        pltpu.SemaphoreType.DMA,
    ]
    cp = pltpu.CompilerParams()
    if "needs_layout_passes" in pltpu.CompilerParams.__dataclass_fields__:
        cp = dataclasses.replace(cp, needs_layout_passes=False)
    k = pl.kernel(_sc_agg_body, out_type=out_type, mesh=mesh,
                  scratch_types=scratch, compiler_params=cp)
    return k(x_lo, x_hi, src3, dst3)


BLK = 1024
GRID = N_PAD // BLK


def _counts_col(parts):
    return jnp.sum(jnp.transpose(parts), axis=1, keepdims=True)


def _mm_bias_body(x_ref, w_ref, b_ref, o_ref):
    o_ref[...] = jnp.dot(x_ref[...], w_ref[...],
                         preferred_element_type=jnp.float32) + b_ref[...]


def _mm_bias(x, wT, b):
    kdim, m = wT.shape
    return pl.pallas_call(
        _mm_bias_body,
        grid=(GRID,),
        in_specs=[
            pl.BlockSpec((BLK, kdim), lambda i: (i, 0)),
            pl.BlockSpec((kdim, m), lambda i: (0, 0)),
            pl.BlockSpec((1, m), lambda i: (0, 0)),
        ],
        out_specs=pl.BlockSpec((BLK, m), lambda i: (i, 0)),
        out_shape=jax.ShapeDtypeStruct((N, m), jnp.float32),
    )(x, wT, b)


def _mm2_bias_body(xa_ref, xb_ref, w_ref, b_ref, o_ref):
    w = w_ref[...]
    o_ref[...] = (
        jnp.dot(xa_ref[...], w[:HALF], preferred_element_type=jnp.float32)
        + jnp.dot(xb_ref[...], w[HALF:], preferred_element_type=jnp.float32)
        + b_ref[...]
    )


def _mm2_bias(xa, xb, wT, b):
    _, m = wT.shape
    return pl.pallas_call(
        _mm2_bias_body,
        grid=(GRID,),
        in_specs=[
            pl.BlockSpec((BLK, HALF), lambda i: (i, 0)),
            pl.BlockSpec((BLK, HALF), lambda i: (i, 0)),
            pl.BlockSpec((2 * HALF, m), lambda i: (0, 0)),
            pl.BlockSpec((1, m), lambda i: (0, 0)),
        ],
        out_specs=pl.BlockSpec((BLK, m), lambda i: (i, 0)),
        out_shape=jax.ShapeDtypeStruct((N, m), jnp.float32),
    )(xa, xb, wT, b)


def _layer_body(alo_ref, ahi_ref, cnt_ref, xr_ref, w_ref, olo_ref, ohi_ref):
    inv = 1.0 / jnp.maximum(_counts_col(cnt_ref[...]), 1.0)
    w = w_ref[...]
    h = (
        jnp.dot(alo_ref[...] * inv, w[:HALF],
                preferred_element_type=jnp.float32)
        + jnp.dot(ahi_ref[...] * inv, w[HALF:],
                  preferred_element_type=jnp.float32)
        + xr_ref[...]
    )
    h = jnp.maximum(h, 0.0)
    olo_ref[...] = h[:, :HALF]
    ohi_ref[...] = h[:, HALF:]


def _layer0(agg_lo, agg_hi, cnt_parts, xr, wlT):
    return pl.pallas_call(
        _layer_body,
        grid=(GRID,),
        in_specs=[
            pl.BlockSpec((BLK, HALF), lambda i: (i, 0)),
            pl.BlockSpec((BLK, HALF), lambda i: (i, 0)),
            pl.BlockSpec((NS, BLK), lambda i: (0, i)),
            pl.BlockSpec((BLK, H), lambda i: (i, 0)),
            pl.BlockSpec((H, H), lambda i: (0, 0)),
        ],
        out_specs=[
            pl.BlockSpec((BLK, HALF), lambda i: (i, 0)),
            pl.BlockSpec((BLK, HALF), lambda i: (i, 0)),
        ],
        out_shape=[
            jax.ShapeDtypeStruct((N, HALF), jnp.float32),
            jax.ShapeDtypeStruct((N, HALF), jnp.float32),
        ],
    )(agg_lo, agg_hi, cnt_parts, xr, wlT)


def _final_body(h0lo_ref, h0hi_ref, alo_ref, ahi_ref, cnt_ref, xr_ref,
                wl_ref, wpa_ref, wpb_ref, bp_ref, o_ref):
    inv = 1.0 / jnp.maximum(_counts_col(cnt_ref[...]), 1.0)
    wl = wl_ref[...]
    h1 = (
        jnp.dot(alo_ref[...] * inv, wl[:HALF],
                preferred_element_type=jnp.float32)
        + jnp.dot(ahi_ref[...] * inv, wl[HALF:],
                  preferred_element_type=jnp.float32)
        + xr_ref[...]
    )
    wpa = wpa_ref[...]
    wpb = wpb_ref[...]
    logits = (
        jnp.dot(h0lo_ref[...], wpa[:HALF], preferred_element_type=jnp.float32)
        + jnp.dot(h0hi_ref[...], wpa[HALF:],
                  preferred_element_type=jnp.float32)
        + jnp.dot(h1[:, :HALF], wpb[:HALF],
                  preferred_element_type=jnp.float32)
        + jnp.dot(h1[:, HALF:], wpb[HALF:],
                  preferred_element_type=jnp.float32)
        + bp_ref[...]
    )
    m = jnp.max(logits, axis=-1, keepdims=True)
    lse = jnp.log(jnp.sum(jnp.exp(logits - m), axis=-1, keepdims=True)) + m
    o_ref[...] = logits - lse


def _final(h0_lo, h0_hi, agg_lo, agg_hi, cnt_parts, xr1, wl1T, wpaT, wpbT,
           bp):
    return pl.pallas_call(
        _final_body,
        grid=(GRID,),
        in_specs=[
            pl.BlockSpec((BLK, HALF), lambda i: (i, 0)),
            pl.BlockSpec((BLK, HALF), lambda i: (i, 0)),
            pl.BlockSpec((BLK, HALF), lambda i: (i, 0)),
            pl.BlockSpec((BLK, HALF), lambda i: (i, 0)),
            pl.BlockSpec((NS, BLK), lambda i: (0, i)),
            pl.BlockSpec((BLK, H), lambda i: (i, 0)),
            pl.BlockSpec((H, H), lambda i: (0, 0)),
            pl.BlockSpec((H, D_OUT), lambda i: (0, 0)),
            pl.BlockSpec((H, D_OUT), lambda i: (0, 0)),
            pl.BlockSpec((1, D_OUT), lambda i: (0, 0)),
        ],
        out_specs=pl.BlockSpec((BLK, D_OUT), lambda i: (i, 0)),
        out_shape=jax.ShapeDtypeStruct((N, D_OUT), jnp.float32),
    )(h0_lo, h0_hi, agg_lo, agg_hi, cnt_parts, xr1, wl1T, wpaT, wpbT, bp)


def _prep_edges(edge_index):
    src = edge_index[0].astype(jnp.int32)
    dst = edge_index[1].astype(jnp.int32)
    pad = E_PAD - E
    r = jnp.arange(pad, dtype=jnp.int32)
    src = jnp.concatenate([src, r % N])
    dst = jnp.concatenate([dst, N + r % (N_PAD - N)])
    return src.reshape(NS, CH, CHUNK), dst.reshape(NS, CH, CHUNK)


def kernel(x, edge_index_0, edge_index_1, W_l0, b_l0, W_r0,
           W_l1, b_l1, W_r1, W_post, b_post):
    f32 = jnp.float32
    src0, dst0 = _prep_edges(edge_index_0)
    src1, dst1 = _prep_edges(edge_index_1)

    x_lo = x[:, :HALF]
    x_hi = x[:, HALF:]

    wl0T = W_l0.T
    wr0T = W_r0.T
    wl1T = W_l1.T
    wr1T = W_r1.T
    wpaT = W_post[:, :H].T
    wpbT = W_post[:, H:].T
    bl0 = b_l0.reshape(1, H)
    bl1 = b_l1.reshape(1, H)
    bp = b_post.reshape(1, D_OUT)

    agg0_lo, agg0_hi, parts0 = _sc_agg(x_lo, x_hi, src0, dst0)
    xr0 = _mm_bias(x, wr0T, bl0)
    h0_lo, h0_hi = _layer0(agg0_lo, agg0_hi, parts0, xr0, wl0T)

    agg1_lo, agg1_hi, parts1 = _sc_agg(h0_lo, h0_hi, src1, dst1)
    xr1 = _mm2_bias(h0_lo, h0_hi, wr1T, bl1)

    return _final(h0_lo, h0_hi, agg1_lo, agg1_hi, parts1, xr1, wl1T,
                  wpaT, wpbT, bp)


# speedup vs baseline: 7.5017x; 1.0725x over previous
import dataclasses
import functools

import jax
import jax.numpy as jnp
from jax import lax
from jax.experimental import pallas as pl
from jax.experimental.pallas import tpu as pltpu
from jax.experimental.pallas import tpu_sc as plsc

N = 10000
E = 160000
D_IN = 256
H = 256
D_OUT = 128

NC = 2
NS = 16
CHUNK = 128
CH = 80
IDXB = 16
E_PAD = NS * CH * CHUNK
N_PAD = 10240
ROWS_PER_SUB = N_PAD // NS
HALF = 128
ZROWS = 128


RED = 128


def _sc_agg_body(xlo_hbm, xhi_hbm, src_hbm, dst_hbm,
                 agglo_hbm, agghi_hbm, parts_hbm,
                 acc_sp, src_v, dst_v, gb0, gb1, cnt_priv,
                 sem0, sem1, sem2, sem3):
    cid = lax.axis_index("c")
    sid = lax.axis_index("s")
    base = sid * ROWS_PER_SUB

    zv = jnp.zeros((16,), jnp.float32)

    @pl.loop(0, CHUNK)
    def _(r):
        @pl.loop(0, HALF, step=16)
        def _(c):
            gb0[r, pl.ds(c, 16)] = zv

    @pl.loop(0, N_PAD, step=16)
    def _(i):
        cnt_priv[pl.ds(i, 16)] = zv

    @pl.loop(0, ROWS_PER_SUB // ZROWS)
    def _(i):
        pltpu.sync_copy(gb0, acc_sp.at[pl.ds(base + i * ZROWS, ZROWS)])

    plsc.subcore_barrier()

    ones_v = jnp.ones((16,), jnp.float32)

    def run(x_hbm, with_counts):
        gbs = (gb0, gb1)
        gsems = (sem0, sem1)
        ssems = (sem2, sem3)

        @pl.loop(0, CH, step=IDXB)
        def _(g):
            pltpu.sync_copy(src_hbm.at[sid, pl.ds(g, IDXB)], src_v)
            pltpu.sync_copy(dst_hbm.at[sid, pl.ds(g, IDXB)], dst_v)
            cps = [None] * IDXB
            scats = [None] * IDXB
            cps[0] = pltpu.async_copy(x_hbm.at[src_v.at[0]], gbs[0],
                                      gsems[0])
            for j in range(IDXB):
                if j + 1 < IDXB:
                    if j >= 1:
                        scats[j - 1].wait()
                    cps[j + 1] = pltpu.async_copy(
                        x_hbm.at[src_v.at[j + 1]], gbs[(j + 1) % 2],
                        gsems[(j + 1) % 2])
                cps[j].wait()
                scats[j] = pltpu.async_copy(
                    gbs[j % 2], acc_sp.at[dst_v.at[j]], ssems[j % 2],
                    add=True)
                if with_counts:
                    for l in range(CHUNK // 16):
                        dvec = dst_v[j, pl.ds(l * 16, 16)]
                        plsc.addupdate_scatter(cnt_priv, [dvec], ones_v)
            scats[IDXB - 2].wait()
            scats[IDXB - 1].wait()

    @pl.when(cid == 0)
    def _():
        run(xlo_hbm, True)

    @pl.when(cid == 1)
    def _():
        run(xhi_hbm, False)

    plsc.subcore_barrier()

    sl = pl.ds(base, ROWS_PER_SUB)

    @pl.when(cid == 1)
    def _():
        pltpu.sync_copy(acc_sp.at[sl], agghi_hbm.at[sl])

    @pl.when(cid == 0)
    def _():
        pltpu.sync_copy(acc_sp.at[sl], agglo_hbm.at[sl])
        pltpu.sync_copy(cnt_priv, parts_hbm.at[sid])


def _sc_agg(x_lo, x_hi, src3, dst3):
    mesh = plsc.VectorSubcoreMesh(core_axis_name="c", subcore_axis_name="s")
    f32 = jnp.float32
    out_type = (
        jax.ShapeDtypeStruct((N_PAD, HALF), f32),
        jax.ShapeDtypeStruct((N_PAD, HALF), f32),
        jax.ShapeDtypeStruct((NS, N_PAD), f32),
    )
    scratch = [
        pltpu.VMEM_SHARED((N_PAD, HALF), f32),
        pltpu.VMEM((IDXB, CHUNK), jnp.int32),
        pltpu.VMEM((IDXB, CHUNK), jnp.int32),
        pltpu.VMEM((CHUNK, HALF), f32),
        pltpu.VMEM((CHUNK, HALF), f32),
        pltpu.VMEM((N_PAD,), f32),
        pltpu.SemaphoreType.DMA,
        pltpu.SemaphoreType.DMA,
        pltpu.SemaphoreType.DMA,
        pltpu.SemaphoreType.DMA,
    ]
    cp = pltpu.CompilerParams()
    if "needs_layout_passes" in pltpu.CompilerParams.__dataclass_fields__:
        cp = dataclasses.replace(cp, needs_layout_passes=False)
    k = pl.kernel(_sc_agg_body, out_type=out_type, mesh=mesh,
                  scratch_types=scratch, compiler_params=cp)
    return k(x_lo, x_hi, src3, dst3)


BLK = 1024
GRID = N_PAD // BLK


def _counts_col(parts):
    return jnp.sum(jnp.transpose(parts), axis=1, keepdims=True)


def _layer_body(alo_ref, ahi_ref, cnt_ref, x_ref, wl_ref, wr_ref, bl_ref,
                olo_ref, ohi_ref):
    inv = 1.0 / jnp.maximum(_counts_col(cnt_ref[...]), 1.0)
    wl = wl_ref[...]
    h = (
        jnp.dot(alo_ref[...] * inv, wl[:HALF],
                preferred_element_type=jnp.float32)
        + jnp.dot(ahi_ref[...] * inv, wl[HALF:],
                  preferred_element_type=jnp.float32)
        + jnp.dot(x_ref[...], wr_ref[...],
                  preferred_element_type=jnp.float32)
        + bl_ref[...]
    )
    h = jnp.maximum(h, 0.0)
    olo_ref[...] = h[:, :HALF]
    ohi_ref[...] = h[:, HALF:]


def _layer0(agg_lo, agg_hi, cnt_parts, x, wlT, wrT, bl):
    return pl.pallas_call(
        _layer_body,
        grid=(GRID,),
        in_specs=[
            pl.BlockSpec((BLK, HALF), lambda i: (i, 0)),
            pl.BlockSpec((BLK, HALF), lambda i: (i, 0)),
            pl.BlockSpec((NS, BLK), lambda i: (0, i)),
            pl.BlockSpec((BLK, H), lambda i: (i, 0)),
            pl.BlockSpec((H, H), lambda i: (0, 0)),
            pl.BlockSpec((H, H), lambda i: (0, 0)),
            pl.BlockSpec((1, H), lambda i: (0, 0)),
        ],
        out_specs=[
            pl.BlockSpec((BLK, HALF), lambda i: (i, 0)),
            pl.BlockSpec((BLK, HALF), lambda i: (i, 0)),
        ],
        out_shape=[
            jax.ShapeDtypeStruct((N, HALF), jnp.float32),
            jax.ShapeDtypeStruct((N, HALF), jnp.float32),
        ],
    )(agg_lo, agg_hi, cnt_parts, x, wlT, wrT, bl)


def _final_body(h0lo_ref, h0hi_ref, alo_ref, ahi_ref, cnt_ref,
                wl_ref, wr_ref, bl_ref, wpa_ref, wpb_ref, bp_ref, o_ref):
    inv = 1.0 / jnp.maximum(_counts_col(cnt_ref[...]), 1.0)
    wl = wl_ref[...]
    wr = wr_ref[...]
    h0lo = h0lo_ref[...]
    h0hi = h0hi_ref[...]
    h1 = (
        jnp.dot(alo_ref[...] * inv, wl[:HALF],
                preferred_element_type=jnp.float32)
        + jnp.dot(ahi_ref[...] * inv, wl[HALF:],
                  preferred_element_type=jnp.float32)
        + jnp.dot(h0lo, wr[:HALF], preferred_element_type=jnp.float32)
        + jnp.dot(h0hi, wr[HALF:], preferred_element_type=jnp.float32)
        + bl_ref[...]
    )
    wpa = wpa_ref[...]
    wpb = wpb_ref[...]
    logits = (
        jnp.dot(h0lo, wpa[:HALF], preferred_element_type=jnp.float32)
        + jnp.dot(h0hi, wpa[HALF:], preferred_element_type=jnp.float32)
        + jnp.dot(h1[:, :HALF], wpb[:HALF],
                  preferred_element_type=jnp.float32)
        + jnp.dot(h1[:, HALF:], wpb[HALF:],
                  preferred_element_type=jnp.float32)
        + bp_ref[...]
    )
    m = jnp.max(logits, axis=-1, keepdims=True)
    lse = jnp.log(jnp.sum(jnp.exp(logits - m), axis=-1, keepdims=True)) + m
    o_ref[...] = logits - lse


def _final(h0_lo, h0_hi, agg_lo, agg_hi, cnt_parts, wl1T, wr1T, bl1,
           wpaT, wpbT, bp):
    return pl.pallas_call(
        _final_body,
        grid=(GRID,),
        in_specs=[
            pl.BlockSpec((BLK, HALF), lambda i: (i, 0)),
            pl.BlockSpec((BLK, HALF), lambda i: (i, 0)),
            pl.BlockSpec((BLK, HALF), lambda i: (i, 0)),
            pl.BlockSpec((BLK, HALF), lambda i: (i, 0)),
            pl.BlockSpec((NS, BLK), lambda i: (0, i)),
            pl.BlockSpec((H, H), lambda i: (0, 0)),
            pl.BlockSpec((H, H), lambda i: (0, 0)),
            pl.BlockSpec((1, H), lambda i: (0, 0)),
            pl.BlockSpec((H, D_OUT), lambda i: (0, 0)),
            pl.BlockSpec((H, D_OUT), lambda i: (0, 0)),
            pl.BlockSpec((1, D_OUT), lambda i: (0, 0)),
        ],
        out_specs=pl.BlockSpec((BLK, D_OUT), lambda i: (i, 0)),
        out_shape=jax.ShapeDtypeStruct((N, D_OUT), jnp.float32),
    )(h0_lo, h0_hi, agg_lo, agg_hi, cnt_parts, wl1T, wr1T, bl1, wpaT,
      wpbT, bp)


def _prep_edges(edge_index):
    src = edge_index[0].astype(jnp.int32)
    dst = edge_index[1].astype(jnp.int32)
    pad = E_PAD - E
    r = jnp.arange(pad, dtype=jnp.int32)
    src = jnp.concatenate([src, r % N])
    dst = jnp.concatenate([dst, N + r % (N_PAD - N)])
    return src.reshape(NS, CH, CHUNK), dst.reshape(NS, CH, CHUNK)


def kernel(x, edge_index_0, edge_index_1, W_l0, b_l0, W_r0,
           W_l1, b_l1, W_r1, W_post, b_post):
    f32 = jnp.float32
    src0, dst0 = _prep_edges(edge_index_0)
    src1, dst1 = _prep_edges(edge_index_1)

    x_lo = x[:, :HALF]
    x_hi = x[:, HALF:]

    wl0T = W_l0.T
    wr0T = W_r0.T
    wl1T = W_l1.T
    wr1T = W_r1.T
    wpaT = W_post[:, :H].T
    wpbT = W_post[:, H:].T
    bl0 = b_l0.reshape(1, H)
    bl1 = b_l1.reshape(1, H)
    bp = b_post.reshape(1, D_OUT)

    agg0_lo, agg0_hi, parts0 = _sc_agg(x_lo, x_hi, src0, dst0)
    h0_lo, h0_hi = _layer0(agg0_lo, agg0_hi, parts0, x, wl0T, wr0T, bl0)

    agg1_lo, agg1_hi, parts1 = _sc_agg(h0_lo, h0_hi, src1, dst1)
    return _final(h0_lo, h0_hi, agg1_lo, agg1_hi, parts1, wl1T, wr1T, bl1,
                  wpaT, wpbT, bp)


# speedup vs baseline: 8.8812x; 1.1839x over previous
import dataclasses
import functools

import jax
import jax.numpy as jnp
from jax import lax
from jax.experimental import pallas as pl
from jax.experimental.pallas import tpu as pltpu
from jax.experimental.pallas import tpu_sc as plsc

N = 10000
E = 160000
D_IN = 256
H = 256
D_OUT = 128

NC = 2
NS = 16
CHUNK = 128
CH = 80
IDXB = 16
E_PAD = NS * CH * CHUNK
N_PAD = 10240
ROWS_PER_SUB = N_PAD // NS
HALF = 128
ZROWS = 128


RED = 128


def _sc_agg_body(xlo_hbm, xhi_hbm, src_hbm, dst_hbm,
                 agglo_hbm, agghi_hbm, parts_hbm,
                 acc_sp, src_v, dst_v, gb0, gb1, cnt_priv,
                 sem0, sem1, sem2, sem3):
    cid = lax.axis_index("c")
    sid = lax.axis_index("s")
    base = sid * ROWS_PER_SUB

    zv = jnp.zeros((16,), jnp.float32)

    @pl.loop(0, CHUNK)
    def _(r):
        @pl.loop(0, HALF, step=16)
        def _(c):
            gb0[r, pl.ds(c, 16)] = zv

    @pl.loop(0, N_PAD, step=16)
    def _(i):
        cnt_priv[pl.ds(i, 16)] = zv

    @pl.loop(0, ROWS_PER_SUB // ZROWS)
    def _(i):
        pltpu.sync_copy(gb0, acc_sp.at[pl.ds(base + i * ZROWS, ZROWS)])

    plsc.subcore_barrier()

    ones_v = jnp.ones((16,), jnp.float32)

    def run(x_hbm, with_counts):
        gbs = (gb0, gb1)
        gsems = (sem0, sem1)
        ssems = (sem2, sem3)

        @pl.loop(0, CH, step=IDXB)
        def _(g):
            pltpu.sync_copy(src_hbm.at[sid, pl.ds(g, IDXB)], src_v)
            pltpu.sync_copy(dst_hbm.at[sid, pl.ds(g, IDXB)], dst_v)
            cps = [None] * IDXB
            scats = [None] * IDXB
            cps[0] = pltpu.async_copy(x_hbm.at[src_v.at[0]], gbs[0],
                                      gsems[0])
            for j in range(IDXB):
                if j + 1 < IDXB:
                    if j >= 1:
                        scats[j - 1].wait()
                    cps[j + 1] = pltpu.async_copy(
                        x_hbm.at[src_v.at[j + 1]], gbs[(j + 1) % 2],
                        gsems[(j + 1) % 2])
                cps[j].wait()
                scats[j] = pltpu.async_copy(
                    gbs[j % 2], acc_sp.at[dst_v.at[j]], ssems[j % 2],
                    add=True)
                if with_counts:
                    for l in range(CHUNK // 16):
                        dvec = dst_v[j, pl.ds(l * 16, 16)]
                        plsc.addupdate_scatter(cnt_priv, [dvec], ones_v)
            scats[IDXB - 2].wait()
            scats[IDXB - 1].wait()

    @pl.when(cid == 0)
    def _():
        run(xlo_hbm, True)

    @pl.when(cid == 1)
    def _():
        run(xhi_hbm, False)

    plsc.subcore_barrier()

    sl = pl.ds(base, ROWS_PER_SUB)

    @pl.when(cid == 1)
    def _():
        pltpu.sync_copy(acc_sp.at[sl], agghi_hbm.at[sl])

    @pl.when(cid == 0)
    def _():
        pltpu.sync_copy(acc_sp.at[sl], agglo_hbm.at[sl])
        pltpu.sync_copy(cnt_priv, parts_hbm.at[sid])


CH1 = E_PAD // (NC * NS * CHUNK)
IDXB1 = 8


def _sc_agg1_body(u_hbm, src_hbm, dst_hbm, agg_a_hbm, agg_b_hbm, parts_hbm,
                  acc_sp, src_v, dst_v, gb0, gb1, cnt_priv,
                  sem0, sem1, sem2, sem3):
    cid = lax.axis_index("c")
    sid = lax.axis_index("s")
    base = sid * ROWS_PER_SUB
    row = cid * NS + sid
    zv = jnp.zeros((16,), jnp.float32)

    @pl.loop(0, CHUNK)
    def _(r):
        @pl.loop(0, HALF, step=16)
        def _(c):
            gb0[r, pl.ds(c, 16)] = zv

    @pl.loop(0, N_PAD, step=16)
    def _(i):
        cnt_priv[pl.ds(i, 16)] = zv

    @pl.loop(0, ROWS_PER_SUB // ZROWS)
    def _(i):
        pltpu.sync_copy(gb0, acc_sp.at[pl.ds(base + i * ZROWS, ZROWS)])

    plsc.subcore_barrier()

    ones_v = jnp.ones((16,), jnp.float32)
    gbs = (gb0, gb1)
    gsems = (sem0, sem1)
    ssems = (sem2, sem3)

    @pl.loop(0, CH1, step=IDXB1)
    def _(g):
        pltpu.sync_copy(src_hbm.at[row, pl.ds(g, IDXB1)], src_v)
        pltpu.sync_copy(dst_hbm.at[row, pl.ds(g, IDXB1)], dst_v)
        cps = [None] * IDXB1
        scats = [None] * IDXB1
        cps[0] = pltpu.async_copy(u_hbm.at[src_v.at[0]], gbs[0], gsems[0])
        for j in range(IDXB1):
            if j + 1 < IDXB1:
                if j >= 1:
                    scats[j - 1].wait()
                cps[j + 1] = pltpu.async_copy(
                    u_hbm.at[src_v.at[j + 1]], gbs[(j + 1) % 2],
                    gsems[(j + 1) % 2])
            cps[j].wait()
            scats[j] = pltpu.async_copy(
                gbs[j % 2], acc_sp.at[dst_v.at[j]], ssems[j % 2], add=True)
            for l in range(CHUNK // 16):
                dvec = dst_v[j, pl.ds(l * 16, 16)]
                plsc.addupdate_scatter(cnt_priv, [dvec], ones_v)
        scats[IDXB1 - 2].wait()
        scats[IDXB1 - 1].wait()

    plsc.subcore_barrier()
    sl = pl.ds(base, ROWS_PER_SUB)
    pltpu.sync_copy(cnt_priv, parts_hbm.at[row])

    @pl.when(cid == 0)
    def _():
        pltpu.sync_copy(acc_sp.at[sl], agg_a_hbm.at[sl])

    @pl.when(cid == 1)
    def _():
        pltpu.sync_copy(acc_sp.at[sl], agg_b_hbm.at[sl])


def _sc_agg1(u1, src3, dst3):
    mesh = plsc.VectorSubcoreMesh(core_axis_name="c", subcore_axis_name="s")
    f32 = jnp.float32
    out_type = (
        jax.ShapeDtypeStruct((N_PAD, HALF), f32),
        jax.ShapeDtypeStruct((N_PAD, HALF), f32),
        jax.ShapeDtypeStruct((NC * NS, N_PAD), f32),
    )
    scratch = [
        pltpu.VMEM_SHARED((N_PAD, HALF), f32),
        pltpu.VMEM((IDXB1, CHUNK), jnp.int32),
        pltpu.VMEM((IDXB1, CHUNK), jnp.int32),
        pltpu.VMEM((CHUNK, HALF), f32),
        pltpu.VMEM((CHUNK, HALF), f32),
        pltpu.VMEM((N_PAD,), f32),
        pltpu.SemaphoreType.DMA,
        pltpu.SemaphoreType.DMA,
        pltpu.SemaphoreType.DMA,
        pltpu.SemaphoreType.DMA,
    ]
    cp = pltpu.CompilerParams()
    if "needs_layout_passes" in pltpu.CompilerParams.__dataclass_fields__:
        cp = dataclasses.replace(cp, needs_layout_passes=False)
    k = pl.kernel(_sc_agg1_body, out_type=out_type, mesh=mesh,
                  scratch_types=scratch, compiler_params=cp)
    return k(u1, src3, dst3)


def _sc_agg(x_lo, x_hi, src3, dst3):
    mesh = plsc.VectorSubcoreMesh(core_axis_name="c", subcore_axis_name="s")
    f32 = jnp.float32
    out_type = (
        jax.ShapeDtypeStruct((N_PAD, HALF), f32),
        jax.ShapeDtypeStruct((N_PAD, HALF), f32),
        jax.ShapeDtypeStruct((NS, N_PAD), f32),
    )
    scratch = [
        pltpu.VMEM_SHARED((N_PAD, HALF), f32),
        pltpu.VMEM((IDXB, CHUNK), jnp.int32),
        pltpu.VMEM((IDXB, CHUNK), jnp.int32),
        pltpu.VMEM((CHUNK, HALF), f32),
        pltpu.VMEM((CHUNK, HALF), f32),
        pltpu.VMEM((N_PAD,), f32),
        pltpu.SemaphoreType.DMA,
        pltpu.SemaphoreType.DMA,
        pltpu.SemaphoreType.DMA,
        pltpu.SemaphoreType.DMA,
    ]
    cp = pltpu.CompilerParams()
    if "needs_layout_passes" in pltpu.CompilerParams.__dataclass_fields__:
        cp = dataclasses.replace(cp, needs_layout_passes=False)
    k = pl.kernel(_sc_agg_body, out_type=out_type, mesh=mesh,
                  scratch_types=scratch, compiler_params=cp)
    return k(x_lo, x_hi, src3, dst3)


BLK = 1024
GRID = N_PAD // BLK


def _counts_col(parts):
    return jnp.sum(jnp.transpose(parts), axis=1, keepdims=True)


def _wfold_body(wl1_ref, wr1_ref, wpa_ref, wpb_ref, bl1_ref, bp_ref,
                wu_ref, wh_ref, ba_ref):
    wpb = wpb_ref[...]
    wu_ref[...] = jnp.dot(wl1_ref[...], wpb,
                          preferred_element_type=jnp.float32)
    wh_ref[...] = wpa_ref[...] + jnp.dot(wr1_ref[...], wpb,
                                         preferred_element_type=jnp.float32)
    ba_ref[...] = bp_ref[...] + jnp.dot(bl1_ref[...], wpb,
                                        preferred_element_type=jnp.float32)


def _wfold(wl1T, wr1T, wpaT, wpbT, bl1, bp):
    full = lambda i: (0, 0)
    return pl.pallas_call(
        _wfold_body,
        grid=(1,),
        in_specs=[
            pl.BlockSpec((H, H), full),
            pl.BlockSpec((H, H), full),
            pl.BlockSpec((H, D_OUT), full),
            pl.BlockSpec((H, D_OUT), full),
            pl.BlockSpec((1, H), full),
            pl.BlockSpec((1, D_OUT), full),
        ],
        out_specs=[
            pl.BlockSpec((H, D_OUT), full),
            pl.BlockSpec((H, D_OUT), full),
            pl.BlockSpec((1, D_OUT), full),
        ],
        out_shape=[
            jax.ShapeDtypeStruct((H, D_OUT), jnp.float32),
            jax.ShapeDtypeStruct((H, D_OUT), jnp.float32),
            jax.ShapeDtypeStruct((1, D_OUT), jnp.float32),
        ],
    )(wl1T, wr1T, wpaT, wpbT, bl1, bp)


def _layer_body(alo_ref, ahi_ref, cnt_ref, x_ref, wl_ref, wr_ref, bl_ref,
                wu_ref, wh_ref, olo_ref, ohi_ref, u1_ref, v0_ref):
    inv = 1.0 / jnp.maximum(_counts_col(cnt_ref[...]), 1.0)
    wl = wl_ref[...]
    h = (
        jnp.dot(alo_ref[...] * inv, wl[:HALF],
                preferred_element_type=jnp.float32)
        + jnp.dot(ahi_ref[...] * inv, wl[HALF:],
                  preferred_element_type=jnp.float32)
        + jnp.dot(x_ref[...], wr_ref[...],
                  preferred_element_type=jnp.float32)
        + bl_ref[...]
    )
    h = jnp.maximum(h, 0.0)
    olo_ref[...] = h[:, :HALF]
    ohi_ref[...] = h[:, HALF:]
    u1_ref[...] = jnp.dot(h, wu_ref[...], preferred_element_type=jnp.float32)
    v0_ref[...] = jnp.dot(h, wh_ref[...], preferred_element_type=jnp.float32)


def _layer0(agg_lo, agg_hi, cnt_parts, x, wlT, wrT, bl, wu, wh):
    return pl.pallas_call(
        _layer_body,
        grid=(GRID,),
        in_specs=[
            pl.BlockSpec((BLK, HALF), lambda i: (i, 0)),
            pl.BlockSpec((BLK, HALF), lambda i: (i, 0)),
            pl.BlockSpec((NS, BLK), lambda i: (0, i)),
            pl.BlockSpec((BLK, H), lambda i: (i, 0)),
            pl.BlockSpec((H, H), lambda i: (0, 0)),
            pl.BlockSpec((H, H), lambda i: (0, 0)),
            pl.BlockSpec((1, H), lambda i: (0, 0)),
            pl.BlockSpec((H, D_OUT), lambda i: (0, 0)),
            pl.BlockSpec((H, D_OUT), lambda i: (0, 0)),
        ],
        out_specs=[
            pl.BlockSpec((BLK, HALF), lambda i: (i, 0)),
            pl.BlockSpec((BLK, HALF), lambda i: (i, 0)),
            pl.BlockSpec((BLK, D_OUT), lambda i: (i, 0)),
            pl.BlockSpec((BLK, D_OUT), lambda i: (i, 0)),
        ],
        out_shape=[
            jax.ShapeDtypeStruct((N, HALF), jnp.float32),
            jax.ShapeDtypeStruct((N, HALF), jnp.float32),
            jax.ShapeDtypeStruct((N, D_OUT), jnp.float32),
            jax.ShapeDtypeStruct((N, D_OUT), jnp.float32),
        ],
    )(agg_lo, agg_hi, cnt_parts, x, wlT, wrT, bl, wu, wh)


def _final_body(v0_ref, agga_ref, aggb_ref, cnt_ref, ba_ref, o_ref):
    inv = 1.0 / jnp.maximum(_counts_col(cnt_ref[...]), 1.0)
    logits = v0_ref[...] + (agga_ref[...] + aggb_ref[...]) * inv + ba_ref[...]
    m = jnp.max(logits, axis=-1, keepdims=True)
    lse = jnp.log(jnp.sum(jnp.exp(logits - m), axis=-1, keepdims=True)) + m
    o_ref[...] = logits - lse


def _final(v0, agg_a, agg_b, cnt_parts, ba):
    return pl.pallas_call(
        _final_body,
        grid=(GRID,),
        in_specs=[
            pl.BlockSpec((BLK, D_OUT), lambda i: (i, 0)),
            pl.BlockSpec((BLK, HALF), lambda i: (i, 0)),
            pl.BlockSpec((BLK, HALF), lambda i: (i, 0)),
            pl.BlockSpec((NC * NS, BLK), lambda i: (0, i)),
            pl.BlockSpec((1, D_OUT), lambda i: (0, 0)),
        ],
        out_specs=pl.BlockSpec((BLK, D_OUT), lambda i: (i, 0)),
        out_shape=jax.ShapeDtypeStruct((N, D_OUT), jnp.float32),
    )(v0, agg_a, agg_b, cnt_parts, ba)


def _prep_edges(edge_index, nsplit=NS):
    src = edge_index[0].astype(jnp.int32)
    dst = edge_index[1].astype(jnp.int32)
    pad = E_PAD - E
    r = jnp.arange(pad, dtype=jnp.int32)
    src = jnp.concatenate([src, r % N])
    dst = jnp.concatenate([dst, N + r % (N_PAD - N)])
    return (src.reshape(nsplit, -1, CHUNK), dst.reshape(nsplit, -1, CHUNK))


def kernel(x, edge_index_0, edge_index_1, W_l0, b_l0, W_r0,
           W_l1, b_l1, W_r1, W_post, b_post):
    f32 = jnp.float32
    src0, dst0 = _prep_edges(edge_index_0)
    src1, dst1 = _prep_edges(edge_index_1, nsplit=NC * NS)

    x_lo = x[:, :HALF]
    x_hi = x[:, HALF:]

    wl0T = W_l0.T
    wr0T = W_r0.T
    wl1T = W_l1.T
    wr1T = W_r1.T
    wpaT = W_post[:, :H].T
    wpbT = W_post[:, H:].T
    bl0 = b_l0.reshape(1, H)
    bl1 = b_l1.reshape(1, H)
    bp = b_post.reshape(1, D_OUT)

    wu, wh, ba = _wfold(wl1T, wr1T, wpaT, wpbT, bl1, bp)
    agg0_lo, agg0_hi, parts0 = _sc_agg(x_lo, x_hi, src0, dst0)
    h0_lo, h0_hi, u1, v0 = _layer0(agg0_lo, agg0_hi, parts0, x, wl0T,
                                   wr0T, bl0, wu, wh)

    agg1_a, agg1_b, parts1 = _sc_agg1(u1, src1, dst1)
    return _final(v0, agg1_a, agg1_b, parts1, ba)


# speedup vs baseline: 8.9727x; 1.0103x over previous
import dataclasses
import functools

import jax
import jax.numpy as jnp
from jax import lax
from jax.experimental import pallas as pl
from jax.experimental.pallas import tpu as pltpu
from jax.experimental.pallas import tpu_sc as plsc

N = 10000
E = 160000
D_IN = 256
H = 256
D_OUT = 128

NC = 2
NS = 16
CHUNK = 128
CH = 80
IDXB = 16
E_PAD = NS * CH * CHUNK
N_PAD = 10240
ROWS_PER_SUB = N_PAD // NS
HALF = 128
ZROWS = 128


RED = 128


def _sc_agg_body(xlo_hbm, xhi_hbm, src_hbm, dst_hbm,
                 agglo_hbm, agghi_hbm, parts_hbm,
                 acc_sp, src_v, dst_v, gb0, gb1, cnt_priv,
                 sem0, sem1, sem2, sem3):
    cid = lax.axis_index("c")
    sid = lax.axis_index("s")
    base = sid * ROWS_PER_SUB

    zv = jnp.zeros((16,), jnp.float32)

    @pl.loop(0, CHUNK)
    def _(r):
        @pl.loop(0, HALF, step=16)
        def _(c):
            gb0[r, pl.ds(c, 16)] = zv

    @pl.loop(0, N_PAD, step=16)
    def _(i):
        cnt_priv[pl.ds(i, 16)] = zv

    @pl.loop(0, ROWS_PER_SUB // ZROWS)
    def _(i):
        pltpu.sync_copy(gb0, acc_sp.at[pl.ds(base + i * ZROWS, ZROWS)])

    plsc.subcore_barrier()

    ones_v = jnp.ones((16,), jnp.float32)

    def run(x_hbm, with_counts):
        gbs = (gb0, gb1)
        gsems = (sem0, sem1)
        ssems = (sem2, sem3)

        @pl.loop(0, CH, step=IDXB)
        def _(g):
            pltpu.sync_copy(src_hbm.at[sid, pl.ds(g, IDXB)], src_v)
            pltpu.sync_copy(dst_hbm.at[sid, pl.ds(g, IDXB)], dst_v)
            cps = [None] * IDXB
            scats = [None] * IDXB
            cps[0] = pltpu.async_copy(x_hbm.at[src_v.at[0]], gbs[0],
                                      gsems[0])
            for j in range(IDXB):
                if j + 1 < IDXB:
                    if j >= 1:
                        scats[j - 1].wait()
                    cps[j + 1] = pltpu.async_copy(
                        x_hbm.at[src_v.at[j + 1]], gbs[(j + 1) % 2],
                        gsems[(j + 1) % 2])
                cps[j].wait()
                scats[j] = pltpu.async_copy(
                    gbs[j % 2], acc_sp.at[dst_v.at[j]], ssems[j % 2],
                    add=True)
                if with_counts:
                    for l in range(CHUNK // 16):
                        dvec = dst_v[j, pl.ds(l * 16, 16)]
                        plsc.addupdate_scatter(cnt_priv, [dvec], ones_v)
            scats[IDXB - 2].wait()
            scats[IDXB - 1].wait()

    @pl.when(cid == 0)
    def _():
        run(xlo_hbm, True)

    @pl.when(cid == 1)
    def _():
        run(xhi_hbm, False)

    plsc.subcore_barrier()

    sl = pl.ds(base, ROWS_PER_SUB)

    @pl.when(cid == 1)
    def _():
        pltpu.sync_copy(acc_sp.at[sl], agghi_hbm.at[sl])

    @pl.when(cid == 0)
    def _():
        pltpu.sync_copy(acc_sp.at[sl], agglo_hbm.at[sl])
        pltpu.sync_copy(cnt_priv, parts_hbm.at[sid])


CH1 = E_PAD // (NC * NS * CHUNK)
IDXB1 = 8


def _sc_agg1_body(u_hbm, src_hbm, dst_hbm, agg_a_hbm, agg_b_hbm, parts_hbm,
                  acc_sp, src_v, dst_v, gb0, gb1, cnt_priv,
                  sem0, sem1, sem2, sem3):
    cid = lax.axis_index("c")
    sid = lax.axis_index("s")
    base = sid * ROWS_PER_SUB
    row = cid * NS + sid
    zv = jnp.zeros((16,), jnp.float32)

    @pl.loop(0, CHUNK)
    def _(r):
        @pl.loop(0, HALF, step=16)
        def _(c):
            gb0[r, pl.ds(c, 16)] = zv

    @pl.loop(0, N_PAD, step=16)
    def _(i):
        cnt_priv[pl.ds(i, 16)] = zv

    @pl.loop(0, ROWS_PER_SUB // ZROWS)
    def _(i):
        pltpu.sync_copy(gb0, acc_sp.at[pl.ds(base + i * ZROWS, ZROWS)])

    plsc.subcore_barrier()

    ones_v = jnp.ones((16,), jnp.float32)
    gbs = (gb0, gb1)
    gsems = (sem0, sem1)
    ssems = (sem2, sem3)

    @pl.loop(0, CH1, step=IDXB1)
    def _(g):
        pltpu.sync_copy(src_hbm.at[row, pl.ds(g, IDXB1)], src_v)
        pltpu.sync_copy(dst_hbm.at[row, pl.ds(g, IDXB1)], dst_v)
        cps = [None] * IDXB1
        scats = [None] * IDXB1
        cps[0] = pltpu.async_copy(u_hbm.at[src_v.at[0]], gbs[0], gsems[0])
        for j in range(IDXB1):
            if j + 1 < IDXB1:
                if j >= 1:
                    scats[j - 1].wait()
                cps[j + 1] = pltpu.async_copy(
                    u_hbm.at[src_v.at[j + 1]], gbs[(j + 1) % 2],
                    gsems[(j + 1) % 2])
            cps[j].wait()
            scats[j] = pltpu.async_copy(
                gbs[j % 2], acc_sp.at[dst_v.at[j]], ssems[j % 2], add=True)
            for l in range(CHUNK // 16):
                dvec = dst_v[j, pl.ds(l * 16, 16)]
                plsc.addupdate_scatter(cnt_priv, [dvec], ones_v)
        scats[IDXB1 - 2].wait()
        scats[IDXB1 - 1].wait()

    plsc.subcore_barrier()
    sl = pl.ds(base, ROWS_PER_SUB)
    pltpu.sync_copy(cnt_priv, parts_hbm.at[row])

    @pl.when(cid == 0)
    def _():
        pltpu.sync_copy(acc_sp.at[sl], agg_a_hbm.at[sl])

    @pl.when(cid == 1)
    def _():
        pltpu.sync_copy(acc_sp.at[sl], agg_b_hbm.at[sl])


def _sc_agg1(u1, src3, dst3):
    mesh = plsc.VectorSubcoreMesh(core_axis_name="c", subcore_axis_name="s")
    f32 = jnp.float32
    out_type = (
        jax.ShapeDtypeStruct((N_PAD, HALF), f32),
        jax.ShapeDtypeStruct((N_PAD, HALF), f32),
        jax.ShapeDtypeStruct((NC * NS, N_PAD), f32),
    )
    scratch = [
        pltpu.VMEM_SHARED((N_PAD, HALF), f32),
        pltpu.VMEM((IDXB1, CHUNK), jnp.int32),
        pltpu.VMEM((IDXB1, CHUNK), jnp.int32),
        pltpu.VMEM((CHUNK, HALF), f32),
        pltpu.VMEM((CHUNK, HALF), f32),
        pltpu.VMEM((N_PAD,), f32),
        pltpu.SemaphoreType.DMA,
        pltpu.SemaphoreType.DMA,
        pltpu.SemaphoreType.DMA,
        pltpu.SemaphoreType.DMA,
    ]
    cp = pltpu.CompilerParams()
    if "needs_layout_passes" in pltpu.CompilerParams.__dataclass_fields__:
        cp = dataclasses.replace(cp, needs_layout_passes=False)
    k = pl.kernel(_sc_agg1_body, out_type=out_type, mesh=mesh,
                  scratch_types=scratch, compiler_params=cp)
    return k(u1, src3, dst3)


def _sc_agg(x_lo, x_hi, src3, dst3):
    mesh = plsc.VectorSubcoreMesh(core_axis_name="c", subcore_axis_name="s")
    f32 = jnp.float32
    out_type = (
        jax.ShapeDtypeStruct((N_PAD, HALF), f32),
        jax.ShapeDtypeStruct((N_PAD, HALF), f32),
        jax.ShapeDtypeStruct((NS, N_PAD), f32),
    )
    scratch = [
        pltpu.VMEM_SHARED((N_PAD, HALF), f32),
        pltpu.VMEM((IDXB, CHUNK), jnp.int32),
        pltpu.VMEM((IDXB, CHUNK), jnp.int32),
        pltpu.VMEM((CHUNK, HALF), f32),
        pltpu.VMEM((CHUNK, HALF), f32),
        pltpu.VMEM((N_PAD,), f32),
        pltpu.SemaphoreType.DMA,
        pltpu.SemaphoreType.DMA,
        pltpu.SemaphoreType.DMA,
        pltpu.SemaphoreType.DMA,
    ]
    cp = pltpu.CompilerParams()
    if "needs_layout_passes" in pltpu.CompilerParams.__dataclass_fields__:
        cp = dataclasses.replace(cp, needs_layout_passes=False)
    k = pl.kernel(_sc_agg_body, out_type=out_type, mesh=mesh,
                  scratch_types=scratch, compiler_params=cp)
    return k(x_lo, x_hi, src3, dst3)


BLK = 1024
GRID = N_PAD // BLK


def _counts_col(parts):
    return jnp.sum(jnp.transpose(parts), axis=1, keepdims=True)


def _wfold_body(wl1_ref, wr1_ref, wpa_ref, wpb_ref, bl1_ref, bp_ref,
                wu_ref, wh_ref, ba_ref):
    wpb = wpb_ref[...]
    wu_ref[...] = jnp.dot(wl1_ref[...], wpb,
                          preferred_element_type=jnp.float32)
    wh_ref[...] = wpa_ref[...] + jnp.dot(wr1_ref[...], wpb,
                                         preferred_element_type=jnp.float32)
    ba_ref[...] = bp_ref[...] + jnp.dot(bl1_ref[...], wpb,
                                        preferred_element_type=jnp.float32)


def _wfold(wl1T, wr1T, wpaT, wpbT, bl1, bp):
    full = lambda i: (0, 0)
    return pl.pallas_call(
        _wfold_body,
        grid=(1,),
        in_specs=[
            pl.BlockSpec((H, H), full),
            pl.BlockSpec((H, H), full),
            pl.BlockSpec((H, D_OUT), full),
            pl.BlockSpec((H, D_OUT), full),
            pl.BlockSpec((1, H), full),
            pl.BlockSpec((1, D_OUT), full),
        ],
        out_specs=[
            pl.BlockSpec((H, D_OUT), full),
            pl.BlockSpec((H, D_OUT), full),
            pl.BlockSpec((1, D_OUT), full),
        ],
        out_shape=[
            jax.ShapeDtypeStruct((H, D_OUT), jnp.float32),
            jax.ShapeDtypeStruct((H, D_OUT), jnp.float32),
            jax.ShapeDtypeStruct((1, D_OUT), jnp.float32),
        ],
    )(wl1T, wr1T, wpaT, wpbT, bl1, bp)


def _layer_body(alo_ref, ahi_ref, cnt_ref, x_ref, wl_ref, wr_ref, bl_ref,
                wu_ref, olo_ref, ohi_ref, u1_ref):
    inv = 1.0 / jnp.maximum(_counts_col(cnt_ref[...]), 1.0)
    wl = wl_ref[...]
    h = (
        jnp.dot(alo_ref[...] * inv, wl[:HALF],
                preferred_element_type=jnp.float32)
        + jnp.dot(ahi_ref[...] * inv, wl[HALF:],
                  preferred_element_type=jnp.float32)
        + jnp.dot(x_ref[...], wr_ref[...],
                  preferred_element_type=jnp.float32)
        + bl_ref[...]
    )
    h = jnp.maximum(h, 0.0)
    olo_ref[...] = h[:, :HALF]
    ohi_ref[...] = h[:, HALF:]
    u1_ref[...] = jnp.dot(h, wu_ref[...], preferred_element_type=jnp.float32)


def _layer0(agg_lo, agg_hi, cnt_parts, x, wlT, wrT, bl, wu):
    return pl.pallas_call(
        _layer_body,
        grid=(GRID,),
        in_specs=[
            pl.BlockSpec((BLK, HALF), lambda i: (i, 0)),
            pl.BlockSpec((BLK, HALF), lambda i: (i, 0)),
            pl.BlockSpec((NS, BLK), lambda i: (0, i)),
            pl.BlockSpec((BLK, H), lambda i: (i, 0)),
            pl.BlockSpec((H, H), lambda i: (0, 0)),
            pl.BlockSpec((H, H), lambda i: (0, 0)),
            pl.BlockSpec((1, H), lambda i: (0, 0)),
            pl.BlockSpec((H, D_OUT), lambda i: (0, 0)),
        ],
        out_specs=[
            pl.BlockSpec((BLK, HALF), lambda i: (i, 0)),
            pl.BlockSpec((BLK, HALF), lambda i: (i, 0)),
            pl.BlockSpec((BLK, D_OUT), lambda i: (i, 0)),
        ],
        out_shape=[
            jax.ShapeDtypeStruct((N, HALF), jnp.float32),
            jax.ShapeDtypeStruct((N, HALF), jnp.float32),
            jax.ShapeDtypeStruct((N, D_OUT), jnp.float32),
        ],
    )(agg_lo, agg_hi, cnt_parts, x, wlT, wrT, bl, wu)


def _v0_body(h0lo_ref, h0hi_ref, wh_ref, o_ref):
    wh = wh_ref[...]
    o_ref[...] = (
        jnp.dot(h0lo_ref[...], wh[:HALF], preferred_element_type=jnp.float32)
        + jnp.dot(h0hi_ref[...], wh[HALF:],
                  preferred_element_type=jnp.float32)
    )


def _v0(h0_lo, h0_hi, wh):
    return pl.pallas_call(
        _v0_body,
        grid=(GRID,),
        in_specs=[
            pl.BlockSpec((BLK, HALF), lambda i: (i, 0)),
            pl.BlockSpec((BLK, HALF), lambda i: (i, 0)),
            pl.BlockSpec((H, D_OUT), lambda i: (0, 0)),
        ],
        out_specs=pl.BlockSpec((BLK, D_OUT), lambda i: (i, 0)),
        out_shape=jax.ShapeDtypeStruct((N, D_OUT), jnp.float32),
    )(h0_lo, h0_hi, wh)


def _final_body(v0_ref, agga_ref, aggb_ref, cnt_ref, ba_ref, o_ref):
    inv = 1.0 / jnp.maximum(_counts_col(cnt_ref[...]), 1.0)
    logits = v0_ref[...] + (agga_ref[...] + aggb_ref[...]) * inv + ba_ref[...]
    m = jnp.max(logits, axis=-1, keepdims=True)
    lse = jnp.log(jnp.sum(jnp.exp(logits - m), axis=-1, keepdims=True)) + m
    o_ref[...] = logits - lse


def _final(v0, agg_a, agg_b, cnt_parts, ba):
    return pl.pallas_call(
        _final_body,
        grid=(GRID,),
        in_specs=[
            pl.BlockSpec((BLK, D_OUT), lambda i: (i, 0)),
            pl.BlockSpec((BLK, HALF), lambda i: (i, 0)),
            pl.BlockSpec((BLK, HALF), lambda i: (i, 0)),
            pl.BlockSpec((NC * NS, BLK), lambda i: (0, i)),
            pl.BlockSpec((1, D_OUT), lambda i: (0, 0)),
        ],
        out_specs=pl.BlockSpec((BLK, D_OUT), lambda i: (i, 0)),
        out_shape=jax.ShapeDtypeStruct((N, D_OUT), jnp.float32),
    )(v0, agg_a, agg_b, cnt_parts, ba)


def _prep_edges(edge_index, nsplit=NS):
    src = edge_index[0].astype(jnp.int32)
    dst = edge_index[1].astype(jnp.int32)
    pad = E_PAD - E
    r = jnp.arange(pad, dtype=jnp.int32)
    src = jnp.concatenate([src, r % N])
    dst = jnp.concatenate([dst, N + r % (N_PAD - N)])
    return (src.reshape(nsplit, -1, CHUNK), dst.reshape(nsplit, -1, CHUNK))


def kernel(x, edge_index_0, edge_index_1, W_l0, b_l0, W_r0,
           W_l1, b_l1, W_r1, W_post, b_post):
    f32 = jnp.float32
    src0, dst0 = _prep_edges(edge_index_0)
    src1, dst1 = _prep_edges(edge_index_1, nsplit=NC * NS)

    x_lo = x[:, :HALF]
    x_hi = x[:, HALF:]

    wl0T = W_l0.T
    wr0T = W_r0.T
    wl1T = W_l1.T
    wr1T = W_r1.T
    wpaT = W_post[:, :H].T
    wpbT = W_post[:, H:].T
    bl0 = b_l0.reshape(1, H)
    bl1 = b_l1.reshape(1, H)
    bp = b_post.reshape(1, D_OUT)

    wu, wh, ba = _wfold(wl1T, wr1T, wpaT, wpbT, bl1, bp)
    agg0_lo, agg0_hi, parts0 = _sc_agg(x_lo, x_hi, src0, dst0)
    h0_lo, h0_hi, u1 = _layer0(agg0_lo, agg0_hi, parts0, x, wl0T,
                               wr0T, bl0, wu)

    agg1_a, agg1_b, parts1 = _sc_agg1(u1, src1, dst1)
    v0 = _v0(h0_lo, h0_hi, wh)
    return _final(v0, agg1_a, agg1_b, parts1, ba)
